# baseline (device time: 61946 ns/iter reference)
import jax
import jax.numpy as jnp
from jax import lax
from jax.experimental import pallas as pl
from jax.experimental.pallas import tpu as pltpu

N_DEV = 8
N_EXP_LOCAL = 2


def kernel(x, router_W, route_idx, expert_W, shared_W):
    n_tok, d_model = x.shape
    n_exp_local, _, d_hidden = expert_W.shape
    n_exp = router_W.shape[1]

    def body(x_ref, router_ref, route_ref, expw_ref, sharedw_ref,
             out_ref, comm_ref, send_sems, recv_sems):
        my = lax.axis_index("i")
        left = lax.rem(my - 1 + N_DEV, N_DEV)
        right = lax.rem(my + 1, N_DEV)

        barrier_sem = pltpu.get_barrier_semaphore()
        for nbr in (left, right):
            pl.semaphore_signal(
                barrier_sem, inc=1,
                device_id=(nbr,), device_id_type=pl.DeviceIdType.MESH,
            )
        pl.semaphore_wait(barrier_sem, 2)

        comm_ref[0] = expw_ref[...].astype(jnp.bfloat16)

        x32 = x_ref[...]
        xb = x32.astype(jnp.bfloat16)

        scores = jnp.dot(x32, router_ref[...], preferred_element_type=jnp.float32)
        s_max = jnp.max(scores, axis=-1, keepdims=True)
        e_s = jnp.exp(scores - s_max)
        probs = e_s / jnp.sum(e_s, axis=-1, keepdims=True)

        exp_iota = lax.broadcasted_iota(jnp.int32, (1, n_exp), 1)
        onehot = (route_ref[...] == exp_iota).astype(jnp.float32)
        gated = probs * onehot

        acc = jnp.dot(xb, sharedw_ref[...].astype(jnp.bfloat16),
                      preferred_element_type=jnp.float32)

        def process(slot, src, acc):
            for k in range(N_EXP_LOCAL):
                e = src * N_EXP_LOCAL + k
                gate_e = jnp.sum(
                    gated * (exp_iota == e).astype(jnp.float32), axis=1
                )
                xw = xb * gate_e.astype(jnp.bfloat16)[:, None]
                acc = acc + jnp.dot(
                    xw, comm_ref[slot, k], preferred_element_type=jnp.float32
                )
            return acc

        for h in range(N_DEV - 1):
            rdma = pltpu.make_async_remote_copy(
                src_ref=comm_ref.at[h],
                dst_ref=comm_ref.at[h + 1],
                send_sem=send_sems.at[h],
                recv_sem=recv_sems.at[h],
                device_id=(right,),
                device_id_type=pl.DeviceIdType.MESH,
            )
            rdma.start()
            src = lax.rem(my - h + N_DEV, N_DEV)
            acc = process(h, src, acc)
            rdma.wait()
        acc = process(N_DEV - 1, lax.rem(my + 1, N_DEV), acc)

        out_ref[...] = acc

    return pl.pallas_call(
        body,
        out_shape=jax.ShapeDtypeStruct((n_tok, d_hidden), jnp.float32),
        in_specs=[pl.BlockSpec(memory_space=pltpu.VMEM)] * 5,
        out_specs=pl.BlockSpec(memory_space=pltpu.VMEM),
        scratch_shapes=[
            pltpu.VMEM((N_DEV, n_exp_local, d_model, d_hidden), jnp.bfloat16),
            pltpu.SemaphoreType.DMA((N_DEV - 1,)),
            pltpu.SemaphoreType.DMA((N_DEV - 1,)),
        ],
        compiler_params=pltpu.CompilerParams(collective_id=0),
    )(x, router_W, route_idx, expert_W, shared_W)


# device time: 46131 ns/iter; 1.3428x vs baseline; 1.3428x over previous
import jax
import jax.numpy as jnp
from jax import lax
from jax.experimental import pallas as pl
from jax.experimental.pallas import tpu as pltpu

N_DEV = 8
N_EXP_LOCAL = 2


def kernel(x, router_W, route_idx, expert_W, shared_W):
    n_tok, d_model = x.shape
    n_exp_local, _, d_hidden = expert_W.shape
    n_exp = router_W.shape[1]

    def body(x_ref, router_ref, route_ref, expw_ref, sharedw_ref,
             out_ref, comm_ref, stage_ref, send_sems, recv_sems):
        my = lax.axis_index("i")

        barrier_sem = pltpu.get_barrier_semaphore()
        for d in range(1, N_DEV):
            pl.semaphore_signal(
                barrier_sem, inc=1,
                device_id=(lax.rem(my + d, N_DEV),),
                device_id_type=pl.DeviceIdType.MESH,
            )
        pl.semaphore_wait(barrier_sem, N_DEV - 1)

        stage_ref[...] = expw_ref[...].astype(jnp.bfloat16)
        sends = []
        for d in range(1, N_DEV):
            tgt = lax.rem(my + d, N_DEV)
            rdma = pltpu.make_async_remote_copy(
                src_ref=stage_ref,
                dst_ref=comm_ref.at[my],
                send_sem=send_sems.at[d - 1],
                recv_sem=recv_sems.at[my],
                device_id=(tgt,),
                device_id_type=pl.DeviceIdType.MESH,
            )
            rdma.start()
            sends.append(rdma)

        x32 = x_ref[...]
        xb = x32.astype(jnp.bfloat16)

        scores = jnp.dot(x32, router_ref[...], preferred_element_type=jnp.float32)
        s_max = jnp.max(scores, axis=-1, keepdims=True)
        e_s = jnp.exp(scores - s_max)
        probs = e_s / jnp.sum(e_s, axis=-1, keepdims=True)

        exp_iota = lax.broadcasted_iota(jnp.int32, (1, n_exp), 1)
        gated = probs * (route_ref[...] == exp_iota).astype(jnp.float32)

        acc = jnp.dot(xb, sharedw_ref[...].astype(jnp.bfloat16),
                      preferred_element_type=jnp.float32)

        def process(w_ref, src, acc):
            for k in range(N_EXP_LOCAL):
                e = src * N_EXP_LOCAL + k
                gate_e = jnp.sum(
                    gated * (exp_iota == e).astype(jnp.float32), axis=1
                )
                xw = xb * gate_e.astype(jnp.bfloat16)[:, None]
                acc = acc + jnp.dot(
                    xw, w_ref[k], preferred_element_type=jnp.float32
                )
            return acc

        acc = process(stage_ref, my, acc)

        for d in range(1, N_DEV):
            src = lax.rem(my + d, N_DEV)
            recv = pltpu.make_async_remote_copy(
                src_ref=stage_ref,
                dst_ref=comm_ref.at[src],
                send_sem=send_sems.at[0],
                recv_sem=recv_sems.at[src],
                device_id=(src,),
                device_id_type=pl.DeviceIdType.MESH,
            )
            recv.wait_recv()
            acc = process(comm_ref.at[src], src, acc)

        out_ref[...] = acc

        for rdma in sends:
            rdma.wait_send()

    return pl.pallas_call(
        body,
        out_shape=jax.ShapeDtypeStruct((n_tok, d_hidden), jnp.float32),
        in_specs=[pl.BlockSpec(memory_space=pltpu.VMEM)] * 5,
        out_specs=pl.BlockSpec(memory_space=pltpu.VMEM),
        scratch_shapes=[
            pltpu.VMEM((N_DEV, n_exp_local, d_model, d_hidden), jnp.bfloat16),
            pltpu.VMEM((n_exp_local, d_model, d_hidden), jnp.bfloat16),
            pltpu.SemaphoreType.DMA((N_DEV - 1,)),
            pltpu.SemaphoreType.DMA((N_DEV,)),
        ],
        compiler_params=pltpu.CompilerParams(collective_id=0),
    )(x, router_W, route_idx, expert_W, shared_W)


# device time: 26885 ns/iter; 2.3041x vs baseline; 1.7159x over previous
import jax
import jax.numpy as jnp
from jax import lax
from jax.experimental import pallas as pl
from jax.experimental.pallas import tpu as pltpu

N_DEV = 8
N_EXP_LOCAL = 2


def kernel(x, router_W, route_idx, expert_W, shared_W):
    n_tok, d_model = x.shape
    n_exp_local, _, d_hidden = expert_W.shape
    n_exp = router_W.shape[1]

    def body(x_ref, router_ref, route_ref, expw_ref, sharedw_ref,
             out_ref, commq_ref, comms_ref, stageq_ref, stages_ref,
             send_sems, recv_sems, send_sems_s, recv_sems_s):
        my = lax.axis_index("i")

        barrier_sem = pltpu.get_barrier_semaphore()
        for d in range(1, N_DEV):
            pl.semaphore_signal(
                barrier_sem, inc=1,
                device_id=(lax.rem(my + d, N_DEV),),
                device_id_type=pl.DeviceIdType.MESH,
            )

        for k in range(N_EXP_LOCAL):
            w = expw_ref[k]
            absmax = jnp.max(jnp.abs(w), axis=0, keepdims=True)
            stageq_ref[k] = jnp.round(w * (127.0 / absmax)).astype(jnp.int8)
            stages_ref[k] = absmax[0] * (1.0 / 127.0)

        pl.semaphore_wait(barrier_sem, N_DEV - 1)

        sends = []
        for d in range(1, N_DEV):
            tgt = lax.rem(my + d, N_DEV)
            rs = pltpu.make_async_remote_copy(
                src_ref=stages_ref,
                dst_ref=comms_ref.at[my],
                send_sem=send_sems_s.at[d - 1],
                recv_sem=recv_sems_s.at[my],
                device_id=(tgt,),
                device_id_type=pl.DeviceIdType.MESH,
            )
            rs.start()
            sends.append(rs)
            for k in range(N_EXP_LOCAL):
                rw = pltpu.make_async_remote_copy(
                    src_ref=stageq_ref.at[k],
                    dst_ref=commq_ref.at[my, k],
                    send_sem=send_sems.at[(d - 1) * N_EXP_LOCAL + k],
                    recv_sem=recv_sems.at[my, k],
                    device_id=(tgt,),
                    device_id_type=pl.DeviceIdType.MESH,
                )
                rw.start()
                sends.append(rw)

        x32 = x_ref[...]
        xb = x32.astype(jnp.bfloat16)

        scores = jnp.dot(x32, router_ref[...], preferred_element_type=jnp.float32)
        s_max = jnp.max(scores, axis=-1, keepdims=True)
        e_s = jnp.exp(scores - s_max)
        probs = e_s / jnp.sum(e_s, axis=-1, keepdims=True)

        exp_iota = lax.broadcasted_iota(jnp.int32, (1, n_exp), 1)
        gated = probs * (route_ref[...] == exp_iota).astype(jnp.float32)

        acc = jnp.dot(xb, sharedw_ref[...].astype(jnp.bfloat16),
                      preferred_element_type=jnp.float32)

        def gate_vec(e):
            return jnp.sum(gated * (exp_iota == e).astype(jnp.float32), axis=1)

        for k in range(N_EXP_LOCAL):
            e = my * N_EXP_LOCAL + k
            xw = xb * gate_vec(e).astype(jnp.bfloat16)[:, None]
            acc = acc + jnp.dot(
                xw, expw_ref[k].astype(jnp.bfloat16),
                preferred_element_type=jnp.float32,
            )

        for d in range(1, N_DEV):
            src = lax.rem(my - d + N_DEV, N_DEV)
            rs = pltpu.make_async_remote_copy(
                src_ref=stages_ref,
                dst_ref=comms_ref.at[src],
                send_sem=send_sems_s.at[0],
                recv_sem=recv_sems_s.at[src],
                device_id=(src,),
                device_id_type=pl.DeviceIdType.MESH,
            )
            rs.wait_recv()
            for k in range(N_EXP_LOCAL):
                rw = pltpu.make_async_remote_copy(
                    src_ref=stageq_ref.at[k],
                    dst_ref=commq_ref.at[src, k],
                    send_sem=send_sems.at[0],
                    recv_sem=recv_sems.at[src, k],
                    device_id=(src,),
                    device_id_type=pl.DeviceIdType.MESH,
                )
                rw.wait_recv()
                e = src * N_EXP_LOCAL + k
                wk = (commq_ref[src, k].astype(jnp.float32)
                      * comms_ref[src, k][None, :]).astype(jnp.bfloat16)
                xw = xb * gate_vec(e).astype(jnp.bfloat16)[:, None]
                acc = acc + jnp.dot(xw, wk, preferred_element_type=jnp.float32)

        out_ref[...] = acc

        for rdma in sends:
            rdma.wait_send()

    return pl.pallas_call(
        body,
        out_shape=jax.ShapeDtypeStruct((n_tok, d_hidden), jnp.float32),
        in_specs=[pl.BlockSpec(memory_space=pltpu.VMEM)] * 5,
        out_specs=pl.BlockSpec(memory_space=pltpu.VMEM),
        scratch_shapes=[
            pltpu.VMEM((N_DEV, n_exp_local, d_model, d_hidden), jnp.int8),
            pltpu.VMEM((N_DEV, n_exp_local, d_hidden), jnp.float32),
            pltpu.VMEM((n_exp_local, d_model, d_hidden), jnp.int8),
            pltpu.VMEM((n_exp_local, d_hidden), jnp.float32),
            pltpu.SemaphoreType.DMA(((N_DEV - 1) * N_EXP_LOCAL,)),
            pltpu.SemaphoreType.DMA((N_DEV, N_EXP_LOCAL)),
            pltpu.SemaphoreType.DMA((N_DEV - 1,)),
            pltpu.SemaphoreType.DMA((N_DEV,)),
        ],
        compiler_params=pltpu.CompilerParams(collective_id=0),
    )(x, router_W, route_idx, expert_W, shared_W)


# device time: 22154 ns/iter; 2.7962x vs baseline; 1.2136x over previous
import jax
import jax.numpy as jnp
from jax import lax
from jax.experimental import pallas as pl
from jax.experimental.pallas import tpu as pltpu

N_DEV = 8
N_EXP_LOCAL = 2
CAP = 64
SLOTS = CAP * N_EXP_LOCAL


def kernel(x, router_W, route_idx, expert_W, shared_W):
    n_tok, d_model = x.shape
    n_exp_local, _, d_hidden = expert_W.shape
    n_exp = router_W.shape[1]

    def body(x_ref, router_ref, route_ref, expw_ref, sharedw_ref,
             out_ref, a_ref, b_ref, rstage_ref, r_ref,
             disp_send_sems, disp_recv_sems, ret_send_sems, ret_recv_sems):
        my = lax.axis_index("i")

        barrier_sem = pltpu.get_barrier_semaphore()
        for d in range(1, N_DEV):
            pl.semaphore_signal(
                barrier_sem, inc=1,
                device_id=(lax.rem(my + d, N_DEV),),
                device_id_type=pl.DeviceIdType.MESH,
            )

        x32 = x_ref[...]
        xb = x32.astype(jnp.bfloat16)
        route = route_ref[...]

        scores = jnp.dot(x32, router_ref[...], preferred_element_type=jnp.float32)
        s_max = jnp.max(scores, axis=-1, keepdims=True)
        e_s = jnp.exp(scores - s_max)
        probs = e_s / jnp.sum(e_s, axis=-1, keepdims=True)
        exp_iota = lax.broadcasted_iota(jnp.int32, (1, n_exp), 1)
        onehot = (route == exp_iota).astype(jnp.float32)
        g_tok = jnp.sum(probs * onehot, axis=1, keepdims=True)
        xg = (x32 * g_tok).astype(jnp.bfloat16)

        iota_r = lax.broadcasted_iota(jnp.int32, (n_tok, n_tok), 0)
        iota_c = lax.broadcasted_iota(jnp.int32, (n_tok, n_tok), 1)
        l_strict = (iota_r > iota_c).astype(jnp.float32)
        rank_all = jnp.dot(l_strict, onehot,
                           preferred_element_type=jnp.float32)

        slot_iota = lax.broadcasted_iota(jnp.int32, (1, SLOTS), 1)
        slot_k = slot_iota // CAP
        slot_c = slot_iota % CAP

        dcols = []
        for d in range(N_DEV):
            e_row = d * N_EXP_LOCAL + slot_k
            e_sel = (lax.broadcasted_iota(jnp.int32, (n_exp, SLOTS), 0)
                     == e_row).astype(jnp.float32)
            rank_sel = jnp.dot(rank_all, e_sel,
                               preferred_element_type=jnp.float32)
            dd = ((route == e_row)
                  & (rank_sel.astype(jnp.int32) == slot_c)
                  ).astype(jnp.bfloat16)
            dcols.append(dd)
            a_blk = lax.dot_general(
                dd, xg, (((0,), (0,)), ((), ())),
                preferred_element_type=jnp.float32)
            a_ref[d] = a_blk.astype(jnp.bfloat16)

        pl.semaphore_wait(barrier_sem, N_DEV - 1)

        for d in range(N_DEV):
            @pl.when(my != d)
            def _(d=d):
                rd = pltpu.make_async_remote_copy(
                    src_ref=a_ref.at[d],
                    dst_ref=b_ref.at[my],
                    send_sem=disp_send_sems.at[d],
                    recv_sem=disp_recv_sems.at[my],
                    device_id=(d,),
                    device_id_type=pl.DeviceIdType.MESH,
                )
                rd.start()

            @pl.when(my == d)
            def _(d=d):
                b_ref[d] = a_ref[d]

        acc = jnp.dot(xb, sharedw_ref[...].astype(jnp.bfloat16),
                      preferred_element_type=jnp.float32)

        w0 = expw_ref[0].astype(jnp.bfloat16)
        w1 = expw_ref[1].astype(jnp.bfloat16)
        row_is_k1 = lax.broadcasted_iota(jnp.int32, (SLOTS, 1), 0) >= CAP
        for s in range(N_DEV):
            @pl.when(my != s)
            def _(s=s):
                rd = pltpu.make_async_remote_copy(
                    src_ref=a_ref.at[s],
                    dst_ref=b_ref.at[s],
                    send_sem=disp_send_sems.at[s],
                    recv_sem=disp_recv_sems.at[s],
                    device_id=(s,),
                    device_id_type=pl.DeviceIdType.MESH,
                )
                rd.wait_recv()

            bs = b_ref[s]
            y0 = jnp.dot(bs, w0, preferred_element_type=jnp.float32)
            y1 = jnp.dot(bs, w1, preferred_element_type=jnp.float32)
            rstage_ref[s] = jnp.where(row_is_k1, y1, y0).astype(jnp.bfloat16)

            @pl.when(my != s)
            def _(s=s):
                rr = pltpu.make_async_remote_copy(
                    src_ref=rstage_ref.at[s],
                    dst_ref=r_ref.at[my],
                    send_sem=ret_send_sems.at[s],
                    recv_sem=ret_recv_sems.at[my],
                    device_id=(s,),
                    device_id_type=pl.DeviceIdType.MESH,
                )
                rr.start()

            @pl.when(my == s)
            def _(s=s):
                r_ref[s] = rstage_ref[s]

        for d in range(N_DEV):
            @pl.when(my != d)
            def _(d=d):
                rr = pltpu.make_async_remote_copy(
                    src_ref=rstage_ref.at[d],
                    dst_ref=r_ref.at[d],
                    send_sem=ret_send_sems.at[d],
                    recv_sem=ret_recv_sems.at[d],
                    device_id=(d,),
                    device_id_type=pl.DeviceIdType.MESH,
                )
                rr.wait_recv()

            acc = acc + jnp.dot(dcols[d], r_ref[d],
                                preferred_element_type=jnp.float32)

        out_ref[...] = acc

        for d in range(N_DEV):
            @pl.when(my != d)
            def _(d=d):
                pltpu.make_async_remote_copy(
                    src_ref=a_ref.at[d],
                    dst_ref=b_ref.at[my],
                    send_sem=disp_send_sems.at[d],
                    recv_sem=disp_recv_sems.at[my],
                    device_id=(d,),
                    device_id_type=pl.DeviceIdType.MESH,
                ).wait_send()
                pltpu.make_async_remote_copy(
                    src_ref=rstage_ref.at[d],
                    dst_ref=r_ref.at[my],
                    send_sem=ret_send_sems.at[d],
                    recv_sem=ret_recv_sems.at[my],
                    device_id=(d,),
                    device_id_type=pl.DeviceIdType.MESH,
                ).wait_send()

    return pl.pallas_call(
        body,
        out_shape=jax.ShapeDtypeStruct((n_tok, d_hidden), jnp.float32),
        in_specs=[pl.BlockSpec(memory_space=pltpu.VMEM)] * 5,
        out_specs=pl.BlockSpec(memory_space=pltpu.VMEM),
        scratch_shapes=[
            pltpu.VMEM((N_DEV, SLOTS, d_model), jnp.bfloat16),
            pltpu.VMEM((N_DEV, SLOTS, d_model), jnp.bfloat16),
            pltpu.VMEM((N_DEV, SLOTS, d_hidden), jnp.bfloat16),
            pltpu.VMEM((N_DEV, SLOTS, d_hidden), jnp.bfloat16),
            pltpu.SemaphoreType.DMA((N_DEV,)),
            pltpu.SemaphoreType.DMA((N_DEV,)),
            pltpu.SemaphoreType.DMA((N_DEV,)),
            pltpu.SemaphoreType.DMA((N_DEV,)),
        ],
        compiler_params=pltpu.CompilerParams(collective_id=0),
    )(x, router_W, route_idx, expert_W, shared_W)


# device time: 21527 ns/iter; 2.8776x vs baseline; 1.0291x over previous
import jax
import jax.numpy as jnp
from jax import lax
from jax.experimental import pallas as pl
from jax.experimental.pallas import tpu as pltpu

N_DEV = 8
N_EXP_LOCAL = 2
CAP = 64
SLOTS = CAP * N_EXP_LOCAL


def kernel(x, router_W, route_idx, expert_W, shared_W):
    n_tok, d_model = x.shape
    n_exp_local, _, d_hidden = expert_W.shape
    n_exp = router_W.shape[1]

    def body(x_ref, router_ref, route_ref, expw_ref, sharedw_ref,
             out_ref, a_ref, b_ref, rstage_ref, r_ref,
             rsc_stage_ref, rsc_ref,
             disp_send_sems, disp_recv_sems, ret_send_sems, ret_recv_sems,
             rsc_send_sems, rsc_recv_sems):
        my = lax.axis_index("i")

        barrier_sem = pltpu.get_barrier_semaphore()
        for d in range(1, N_DEV):
            pl.semaphore_signal(
                barrier_sem, inc=1,
                device_id=(lax.rem(my + d, N_DEV),),
                device_id_type=pl.DeviceIdType.MESH,
            )

        x32 = x_ref[...]
        xb = x32.astype(jnp.bfloat16)
        route = route_ref[...]

        scores = jnp.dot(x32, router_ref[...], preferred_element_type=jnp.float32)
        s_max = jnp.max(scores, axis=-1, keepdims=True)
        e_s = jnp.exp(scores - s_max)
        probs = e_s / jnp.sum(e_s, axis=-1, keepdims=True)
        exp_iota = lax.broadcasted_iota(jnp.int32, (1, n_exp), 1)
        onehot = (route == exp_iota).astype(jnp.float32)
        g_tok = jnp.sum(probs * onehot, axis=1, keepdims=True)
        xg = (x32 * g_tok).astype(jnp.bfloat16)

        iota_r = lax.broadcasted_iota(jnp.int32, (n_tok, n_tok), 0)
        iota_c = lax.broadcasted_iota(jnp.int32, (n_tok, n_tok), 1)
        l_strict = (iota_r > iota_c).astype(jnp.float32)
        rank_all = jnp.dot(l_strict, onehot,
                           preferred_element_type=jnp.float32)

        slot_iota = lax.broadcasted_iota(jnp.int32, (1, SLOTS), 1)
        slot_k = slot_iota // CAP
        slot_c = slot_iota % CAP

        dcols = []
        for d in range(N_DEV):
            e_row = d * N_EXP_LOCAL + slot_k
            e_sel = (lax.broadcasted_iota(jnp.int32, (n_exp, SLOTS), 0)
                     == e_row).astype(jnp.float32)
            rank_sel = jnp.dot(rank_all, e_sel,
                               preferred_element_type=jnp.float32)
            dd = ((route == e_row)
                  & (rank_sel.astype(jnp.int32) == slot_c)
                  ).astype(jnp.bfloat16)
            dcols.append(dd)
            a_blk = lax.dot_general(
                dd, xg, (((0,), (0,)), ((), ())),
                preferred_element_type=jnp.float32)
            a_ref[d] = a_blk.astype(jnp.bfloat16)

        pl.semaphore_wait(barrier_sem, N_DEV - 1)

        for d in range(N_DEV):
            @pl.when(my != d)
            def _(d=d):
                rd = pltpu.make_async_remote_copy(
                    src_ref=a_ref.at[d],
                    dst_ref=b_ref.at[my],
                    send_sem=disp_send_sems.at[d],
                    recv_sem=disp_recv_sems.at[my],
                    device_id=(d,),
                    device_id_type=pl.DeviceIdType.MESH,
                )
                rd.start()

            @pl.when(my == d)
            def _(d=d):
                b_ref[d] = a_ref[d]

        acc = jnp.dot(xb, sharedw_ref[...].astype(jnp.bfloat16),
                      preferred_element_type=jnp.float32)

        w0 = expw_ref[0].astype(jnp.bfloat16)
        w1 = expw_ref[1].astype(jnp.bfloat16)
        row_is_k1 = lax.broadcasted_iota(jnp.int32, (SLOTS, 1), 0) >= CAP
        for s in range(N_DEV):
            @pl.when(my != s)
            def _(s=s):
                rd = pltpu.make_async_remote_copy(
                    src_ref=a_ref.at[s],
                    dst_ref=b_ref.at[s],
                    send_sem=disp_send_sems.at[s],
                    recv_sem=disp_recv_sems.at[s],
                    device_id=(s,),
                    device_id_type=pl.DeviceIdType.MESH,
                )
                rd.wait_recv()

            bs = b_ref[s]
            y0 = jnp.dot(bs, w0, preferred_element_type=jnp.float32)
            y1 = jnp.dot(bs, w1, preferred_element_type=jnp.float32)
            ys = jnp.where(row_is_k1, y1, y0)
            absmax = jnp.maximum(
                jnp.max(jnp.abs(ys), axis=1, keepdims=True), 1e-20)
            rstage_ref[s] = jnp.round(ys * (127.0 / absmax)).astype(jnp.int8)
            rsc_stage_ref[s] = absmax * (1.0 / 127.0)

            @pl.when(my != s)
            def _(s=s):
                rr = pltpu.make_async_remote_copy(
                    src_ref=rstage_ref.at[s],
                    dst_ref=r_ref.at[my],
                    send_sem=ret_send_sems.at[s],
                    recv_sem=ret_recv_sems.at[my],
                    device_id=(s,),
                    device_id_type=pl.DeviceIdType.MESH,
                )
                rr.start()
                rc = pltpu.make_async_remote_copy(
                    src_ref=rsc_stage_ref.at[s],
                    dst_ref=rsc_ref.at[my],
                    send_sem=rsc_send_sems.at[s],
                    recv_sem=rsc_recv_sems.at[my],
                    device_id=(s,),
                    device_id_type=pl.DeviceIdType.MESH,
                )
                rc.start()

            @pl.when(my == s)
            def _(s=s):
                r_ref[s] = rstage_ref[s]
                rsc_ref[s] = rsc_stage_ref[s]

        for d in range(N_DEV):
            @pl.when(my != d)
            def _(d=d):
                rr = pltpu.make_async_remote_copy(
                    src_ref=rstage_ref.at[d],
                    dst_ref=r_ref.at[d],
                    send_sem=ret_send_sems.at[d],
                    recv_sem=ret_recv_sems.at[d],
                    device_id=(d,),
                    device_id_type=pl.DeviceIdType.MESH,
                )
                rr.wait_recv()
                rc = pltpu.make_async_remote_copy(
                    src_ref=rsc_stage_ref.at[d],
                    dst_ref=rsc_ref.at[d],
                    send_sem=rsc_send_sems.at[d],
                    recv_sem=rsc_recv_sems.at[d],
                    device_id=(d,),
                    device_id_type=pl.DeviceIdType.MESH,
                )
                rc.wait_recv()

            rd_b = (r_ref[d].astype(jnp.float32)
                    * rsc_ref[d]).astype(jnp.bfloat16)
            acc = acc + jnp.dot(dcols[d], rd_b,
                                preferred_element_type=jnp.float32)

        out_ref[...] = acc

        for d in range(N_DEV):
            @pl.when(my != d)
            def _(d=d):
                pltpu.make_async_remote_copy(
                    src_ref=a_ref.at[d],
                    dst_ref=b_ref.at[my],
                    send_sem=disp_send_sems.at[d],
                    recv_sem=disp_recv_sems.at[my],
                    device_id=(d,),
                    device_id_type=pl.DeviceIdType.MESH,
                ).wait_send()
                pltpu.make_async_remote_copy(
                    src_ref=rstage_ref.at[d],
                    dst_ref=r_ref.at[my],
                    send_sem=ret_send_sems.at[d],
                    recv_sem=ret_recv_sems.at[my],
                    device_id=(d,),
                    device_id_type=pl.DeviceIdType.MESH,
                ).wait_send()
                pltpu.make_async_remote_copy(
                    src_ref=rsc_stage_ref.at[d],
                    dst_ref=rsc_ref.at[my],
                    send_sem=rsc_send_sems.at[d],
                    recv_sem=rsc_recv_sems.at[my],
                    device_id=(d,),
                    device_id_type=pl.DeviceIdType.MESH,
                ).wait_send()

    return pl.pallas_call(
        body,
        out_shape=jax.ShapeDtypeStruct((n_tok, d_hidden), jnp.float32),
        in_specs=[pl.BlockSpec(memory_space=pltpu.VMEM)] * 5,
        out_specs=pl.BlockSpec(memory_space=pltpu.VMEM),
        scratch_shapes=[
            pltpu.VMEM((N_DEV, SLOTS, d_model), jnp.bfloat16),
            pltpu.VMEM((N_DEV, SLOTS, d_model), jnp.bfloat16),
            pltpu.VMEM((N_DEV, SLOTS, d_hidden), jnp.int8),
            pltpu.VMEM((N_DEV, SLOTS, d_hidden), jnp.int8),
            pltpu.VMEM((N_DEV, SLOTS, 1), jnp.float32),
            pltpu.VMEM((N_DEV, SLOTS, 1), jnp.float32),
            pltpu.SemaphoreType.DMA((N_DEV,)),
            pltpu.SemaphoreType.DMA((N_DEV,)),
            pltpu.SemaphoreType.DMA((N_DEV,)),
            pltpu.SemaphoreType.DMA((N_DEV,)),
            pltpu.SemaphoreType.DMA((N_DEV,)),
            pltpu.SemaphoreType.DMA((N_DEV,)),
        ],
        compiler_params=pltpu.CompilerParams(collective_id=0),
    )(x, router_W, route_idx, expert_W, shared_W)
